# baseline (speedup 1.0000x reference)
import jax
import jax.numpy as jnp
from jax import lax
from jax.experimental import pallas as pl
from jax.experimental.pallas import tpu as pltpu
from jax.experimental.pallas import tpu_sc as plsc

VOCAB = 1000000
D = 64
B = 16384
L = 20

NC = 2
NS = 16
LANES = 16
NW = NC * NS

NBUK = VOCAB // 128 + 1
BUK_PER_W = 7808 // NW
PAIR_ROWS = VOCAB // 2

BPW = B // NW
C = 16
NCH = BPW // C
IDX_LEN = BPW * L
RPC = C * L
GSLICES = ((0, 128), (128, 128), (256, 64))


def _newton_sqrt(x):
    xs = jnp.maximum(x, jnp.float32(1e-30))
    i = plsc.bitcast(xs, jnp.int32)
    i = jnp.int32(0x5F3759DF) - (i >> 1)
    y = plsc.bitcast(i, jnp.float32)
    half = jnp.float32(0.5)
    threehalf = jnp.float32(1.5)
    for _ in range(3):
        y = y * (threehalf - half * xs * y * y)
    return xs * y


_LANE = tuple(range(LANES))


def _detile_body(wt_hbm, wtail_hbm, wp_hbm, tin_v, pb_v, sem_in, sem_out):
    cid = lax.axis_index("c")
    sid = lax.axis_index("s")
    w = sid * NC + cid
    start = w * BUK_PER_W

    lane = jnp.arange(LANES, dtype=jnp.int32)
    rowc = []
    colc = [(lane & jnp.int32(1)) << 6] * 8
    for v in range(8):
        rowc.append((lane >> 1) + jnp.int32(8 * v))

    def issue_in(j, p):
        co = pl.multiple_of(128 * j, 128)
        for i in range(8):
            pltpu.async_copy(
                wt_hbm.at[pl.ds(8 * i, 8), pl.ds(co, 128)],
                tin_v.at[p, i], sem_in.at[p])

    def drain_in(j, p):
        co = pl.multiple_of(128 * j, 128)
        for i in range(8):
            pltpu.make_async_copy(
                wt_hbm.at[pl.ds(8 * i, 8), pl.ds(co, 128)],
                tin_v.at[p, i], sem_in.at[p]).wait()

    def issue_out(j, p):
        ro = pl.multiple_of(64 * j, 64)
        pltpu.async_copy(pb_v.at[p], wp_hbm.at[pl.ds(ro, 64)],
                         sem_out.at[p])

    def drain_out(j, p):
        ro = pl.multiple_of(64 * j, 64)
        pltpu.make_async_copy(pb_v.at[p], wp_hbm.at[pl.ds(ro, 64)],
                              sem_out.at[p]).wait()

    issue_in(start, 0)

    def bbody(jj, carry):
        j = start + jj
        p = lax.rem(jj, 2)

        @pl.when(jj + 1 < BUK_PER_W)
        def _():
            issue_in(j + 1, 1 - p)

        drain_in(j, p)

        @pl.when(jj >= 2)
        def _():
            drain_out(j - 2, p)

        for i in range(8):
            for dd in range(8):
                dcol = jnp.int32(8 * i + dd)
                for v in range(8):
                    vals = tin_v[p, i, dd, pl.ds(16 * v, 16)]
                    plsc.store_scatter(
                        pb_v.at[p], [rowc[v], colc[v] + dcol], vals)
        issue_out(j, p)
        return carry

    lax.fori_loop(0, BUK_PER_W, bbody, 0)
    drain_out(start + BUK_PER_W - 2, lax.rem(BUK_PER_W - 2, 2))
    drain_out(start + BUK_PER_W - 1, lax.rem(BUK_PER_W - 1, 2))

    @pl.when(w < 4)
    def _():
        j = 7808 + w
        issue_in(j, 0)
        drain_in(j, 0)
        for i in range(8):
            for dd in range(8):
                dcol = jnp.int32(8 * i + dd)
                for v in range(8):
                    vals = tin_v[0, i, dd, pl.ds(16 * v, 16)]
                    plsc.store_scatter(
                        pb_v.at[0], [rowc[v], colc[v] + dcol], vals)
        issue_out(j, 0)
        drain_out(j, 0)

    @pl.when(w == 4)
    def _():
        pltpu.async_copy(wtail_hbm, wp_hbm.at[pl.ds(PAIR_ROWS - 32, 32)],
                         sem_out.at[0])
        pltpu.make_async_copy(wtail_hbm,
                              wp_hbm.at[pl.ds(PAIR_ROWS - 32, 32)],
                              sem_out.at[0]).wait()


def _cbow_body(wp_hbm, ctxt_hbm, tgt_hbm, out_hbm,
               idx_v, par_v, ctxt_v, tgt2_v, tgtp_v, tpar_v, w0_v,
               rows_v, trow_v,
               n0_v, red_v, stage_v, shared_v,
               sem_rows, sem_tgt):
    cid = lax.axis_index("c")
    sid = lax.axis_index("s")
    w = sid * NC + cid

    cbase = pl.multiple_of(w * BPW, BPW)
    for k in range(BPW // 128):
        pltpu.sync_copy(ctxt_hbm.at[:, pl.ds(cbase + 128 * k, 128)],
                        ctxt_v.at[:, pl.ds(128 * k, 128)])
    tbase = pl.multiple_of((w // 2) * 8, 8)
    pltpu.sync_copy(tgt_hbm.at[pl.ds(tbase, 8)], tgt2_v)
    pltpu.sync_copy(wp_hbm.at[0], w0_v)
    h4 = lax.rem(w, 2) * 4

    lane = jnp.arange(LANES, dtype=jnp.int32)

    def tbody(g, carry):
        base = (g * LANES + lane) * L
        for l in range(L):
            ids = ctxt_v[l, pl.ds(g * LANES, LANES)]
            plsc.store_scatter(idx_v, [base + l], ids >> 1)
            plsc.store_scatter(par_v, [base + l], (ids & 1) << 6)
        return carry

    lax.fori_loop(0, BPW // LANES, tbody, 0)

    for g in range(BPW // LANES):
        ids = tgt2_v[h4 + g // 8, pl.ds((g % 8) * LANES, LANES)]
        sl = pl.ds(g * LANES, LANES)
        tgtp_v[sl] = ids >> 1
        tpar_v[sl] = (ids & 1) << 6

    eps = jnp.float32(1e-8)

    def issue(c, p):
        for off, n in GSLICES:
            pltpu.async_copy(
                wp_hbm.at[idx_v.at[pl.ds(c * RPC + off, n)]],
                rows_v.at[p, pl.ds(off, n)], sem_rows.at[p])
        pltpu.async_copy(
            wp_hbm.at[tgtp_v.at[pl.ds(c * C, C)]], trow_v.at[p],
            sem_tgt.at[p])

    def drain(c, p):
        for off, n in GSLICES:
            pltpu.make_async_copy(
                wp_hbm.at[idx_v.at[pl.ds(c * RPC + off, n)]],
                rows_v.at[p, pl.ds(off, n)], sem_rows.at[p]).wait()
        pltpu.make_async_copy(
            wp_hbm.at[tgtp_v.at[pl.ds(c * C, C)]], trow_v.at[p],
            sem_tgt.at[p]).wait()

    issue(0, 0)

    def chunk_body(c, cos_sum):
        p = lax.rem(c, 2)

        @pl.when(c + 1 < NCH)
        def _():
            issue(c + 1, 1 - p)

        drain(c, p)

        cnt = jnp.zeros((LANES,), jnp.int32)
        for l in range(L):
            ids = ctxt_v[l, pl.ds(c * C, LANES)]
            cnt = cnt + jnp.where(ids == jnp.int32(0),
                                  jnp.int32(1), jnp.int32(0))
        n0_v[pl.ds(0, LANES)] = cnt.astype(jnp.float32)

        def ebody(e, acc_cos):
            fo = c * RPC + e * L
            h0 = par_v[pl.ds(fo, LANES)][0]
            acc = [rows_v[p, e * L, pl.ds(h0 + 16 * d, 16)]
                   for d in range(4)]
            for l in range(1, L):
                hl = par_v[pl.ds(fo + l, LANES)][0]
                for d in range(4):
                    acc[d] = acc[d] + rows_v[p, e * L + l,
                                             pl.ds(hl + 16 * d, 16)]
            n0 = jnp.full((LANES,), n0_v[pl.ds(e, LANES)][0], jnp.float32)
            inv = jnp.float32(1.0) / (jnp.float32(L) - n0 + jnp.float32(1e-6))
            pooled = [(acc[d] - n0 * w0_v[pl.ds(16 * d, 16)]) * inv
                      for d in range(4)]
            ht = tpar_v[pl.ds(c * C + e, LANES)][0]
            tg = [trow_v[p, e, pl.ds(ht + 16 * d, 16)] for d in range(4)]
            dv = pooled[0] * tg[0]
            av = pooled[0] * pooled[0]
            bv = tg[0] * tg[0]
            for d in range(1, 4):
                dv = dv + pooled[d] * tg[d]
                av = av + pooled[d] * pooled[d]
                bv = bv + tg[d] * tg[d]
            dots = jnp.full((LANES,), jnp.sum(dv), jnp.float32)
            na2 = jnp.full((LANES,), jnp.sum(av), jnp.float32)
            nb2 = jnp.full((LANES,), jnp.sum(bv), jnp.float32)
            na = _newton_sqrt(na2)
            nb = _newton_sqrt(nb2)
            cos = dots / (jnp.maximum(na, eps) * jnp.maximum(nb, eps))
            return acc_cos + cos

        return lax.fori_loop(0, C, ebody, cos_sum)

    total = lax.fori_loop(0, NCH, chunk_body,
                          jnp.zeros((LANES,), jnp.float32))

    z = jnp.zeros((LANES,), jnp.float32)
    for r in range(8):
        for k in range(8):
            stage_v[r, pl.ds(16 * k, 16)] = z
    stage_v[0, pl.ds(0, LANES)] = total
    pltpu.sync_copy(stage_v, shared_v.at[sid])
    plsc.subcore_barrier()

    @pl.when(sid == 0)
    def _():
        tot = jnp.zeros((LANES,), jnp.float32)
        for s in range(NS):
            pltpu.sync_copy(shared_v.at[s], red_v)
            tot = tot + red_v[0, pl.ds(0, LANES)]
        stage_v[0, pl.ds(0, LANES)] = -tot / jnp.float32(B)
        pltpu.sync_copy(stage_v, out_hbm.at[cid])


@jax.jit
def _cbow_loss(ctxt, tgt2d, Wt, wtail):
    mesh = plsc.VectorSubcoreMesh(
        core_axis_name="c", subcore_axis_name="s",
        num_cores=NC, num_subcores=NS)
    cp = pltpu.CompilerParams(
        needs_layout_passes=False, use_tc_tiling_on_sc=True)
    wpair = pl.kernel(
        _detile_body,
        out_type=jax.ShapeDtypeStruct((PAIR_ROWS, 128), jnp.float32),
        mesh=mesh,
        compiler_params=cp,
        scratch_types=[
            pltpu.VMEM((2, 8, 8, 128), jnp.float32),
            pltpu.VMEM((2, 64, 128), jnp.float32),
            pltpu.SemaphoreType.DMA((2,)),
            pltpu.SemaphoreType.DMA((2,)),
        ],
    )(Wt, wtail)
    partial = pl.kernel(
        _cbow_body,
        out_type=jax.ShapeDtypeStruct((NC, 8, 128), jnp.float32),
        mesh=mesh,
        compiler_params=cp,
        scratch_types=[
            pltpu.VMEM((IDX_LEN,), jnp.int32),
            pltpu.VMEM((IDX_LEN + LANES,), jnp.int32),
            pltpu.VMEM((L, BPW), jnp.int32),
            pltpu.VMEM((8, 128), jnp.int32),
            pltpu.VMEM((BPW,), jnp.int32),
            pltpu.VMEM((BPW + LANES,), jnp.int32),
            pltpu.VMEM((128,), jnp.float32),
            pltpu.VMEM((2, RPC, 128), jnp.float32),
            pltpu.VMEM((2, C, 128), jnp.float32),
            pltpu.VMEM((LANES + LANES,), jnp.float32),
            pltpu.VMEM((8, 128), jnp.float32),
            pltpu.VMEM((8, 128), jnp.float32),
            pltpu.VMEM_SHARED((NS, 8, 128), jnp.float32),
            pltpu.SemaphoreType.DMA((2,)),
            pltpu.SemaphoreType.DMA((2,)),
        ],
    )(wpair, ctxt, tgt2d)
    return partial[0, 0, 0] + partial[1, 0, 0]


def kernel(contexts, targets, W):
    wtail = W[VOCAB - 64:].reshape(32, 128)
    return _cbow_loss(contexts.T, targets.reshape(128, 128), W.T, wtail)


# speedup vs baseline: 1.0901x; 1.0901x over previous
import jax
import jax.numpy as jnp
from jax import lax
from jax.experimental import pallas as pl
from jax.experimental.pallas import tpu as pltpu
from jax.experimental.pallas import tpu_sc as plsc

VOCAB = 1000000
D = 64
B = 16384
L = 20

NC = 2
NS = 16
LANES = 16
NW = NC * NS

NBUK = VOCAB // 128 + 1
BUK_PER_W = 7808 // NW
PAIR_ROWS = VOCAB // 2

BPW = B // NW
C = 32
NCH = BPW // C
IDX_LEN = BPW * L
ROWS_PER_CHUNK = C * L
GATHERS = ROWS_PER_CHUNK // 128


def _newton_sqrt(x):
    xs = jnp.maximum(x, jnp.float32(1e-30))
    i = plsc.bitcast(xs, jnp.int32)
    i = jnp.int32(0x5F3759DF) - (i >> 1)
    y = plsc.bitcast(i, jnp.float32)
    half = jnp.float32(0.5)
    threehalf = jnp.float32(1.5)
    for _ in range(3):
        y = y * (threehalf - half * xs * y * y)
    return xs * y


_LANE = tuple(range(LANES))


def _detile_body(wt_hbm, wtail_hbm, wp_hbm, tin_v, pb_v, sem_in, sem_out):
    cid = lax.axis_index("c")
    sid = lax.axis_index("s")
    w = sid * NC + cid
    start = w * BUK_PER_W

    lane = jnp.arange(LANES, dtype=jnp.int32)
    rowc = []
    colc = [(lane & jnp.int32(1)) << 6] * 8
    for v in range(8):
        rowc.append((lane >> 1) + jnp.int32(8 * v))

    def issue_in(j, p):
        co = pl.multiple_of(128 * j, 128)
        pltpu.async_copy(wt_hbm.at[:, pl.ds(co, 128)], tin_v.at[p],
                         sem_in.at[p])

    def drain_in(j, p):
        co = pl.multiple_of(128 * j, 128)
        pltpu.make_async_copy(wt_hbm.at[:, pl.ds(co, 128)], tin_v.at[p],
                              sem_in.at[p]).wait()

    def issue_out(j, p):
        ro = pl.multiple_of(64 * j, 64)
        pltpu.async_copy(pb_v.at[p], wp_hbm.at[pl.ds(ro, 64)],
                         sem_out.at[p])

    def drain_out(j, p):
        ro = pl.multiple_of(64 * j, 64)
        pltpu.make_async_copy(pb_v.at[p], wp_hbm.at[pl.ds(ro, 64)],
                              sem_out.at[p]).wait()

    issue_in(start, 0)

    def bbody(jj, carry):
        j = start + jj
        p = lax.rem(jj, 2)

        @pl.when(jj + 1 < BUK_PER_W)
        def _():
            issue_in(j + 1, 1 - p)

        drain_in(j, p)

        @pl.when(jj >= 2)
        def _():
            drain_out(j - 2, p)

        for d in range(64):
            dcol = jnp.int32(d)
            for v in range(8):
                vals = tin_v[p, d, pl.ds(16 * v, 16)]
                plsc.store_scatter(
                    pb_v.at[p], [rowc[v], colc[v] + dcol], vals)
        issue_out(j, p)
        return carry

    lax.fori_loop(0, BUK_PER_W, bbody, 0)
    drain_out(start + BUK_PER_W - 2, lax.rem(BUK_PER_W - 2, 2))
    drain_out(start + BUK_PER_W - 1, lax.rem(BUK_PER_W - 1, 2))

    @pl.when(w < 4)
    def _():
        j = 7808 + w
        issue_in(j, 0)
        drain_in(j, 0)
        for d in range(64):
            dcol = jnp.int32(d)
            for v in range(8):
                vals = tin_v[0, d, pl.ds(16 * v, 16)]
                plsc.store_scatter(
                    pb_v.at[0], [rowc[v], colc[v] + dcol], vals)
        issue_out(j, 0)
        drain_out(j, 0)

    @pl.when(w == 4)
    def _():
        pltpu.async_copy(wtail_hbm, wp_hbm.at[pl.ds(PAIR_ROWS - 32, 32)],
                         sem_out.at[0])
        pltpu.make_async_copy(wtail_hbm,
                              wp_hbm.at[pl.ds(PAIR_ROWS - 32, 32)],
                              sem_out.at[0]).wait()


def _cbow_body(w_hbm, ctxt_hbm, tgt_hbm, out_hbm,
          idx_v, ctxt_v, tgt_v, w0_v, rows_v, trow_v,
          n0_v, cacc_v, red_v, stage_v, shared_v,
          sem_rows, sem_tgt):
    cid = lax.axis_index("c")
    sid = lax.axis_index("s")
    w = sid * NC + cid

    pltpu.sync_copy(ctxt_hbm.at[:, pl.ds(w * BPW, BPW)], ctxt_v)
    pltpu.sync_copy(tgt_hbm.at[pl.ds(w * BPW, BPW)], tgt_v)
    pltpu.sync_copy(w_hbm.at[0], w0_v)

    lane = jnp.arange(LANES, dtype=jnp.int32)

    def tbody(g, carry):
        base = (g * LANES + lane) * L
        for l in range(L):
            ids = ctxt_v[l, pl.ds(g * LANES, LANES)]
            plsc.store_scatter(idx_v, [base + l], ids)
        return carry

    lax.fori_loop(0, BPW // LANES, tbody, 0)

    eps = jnp.float32(1e-8)

    def issue(c, p):
        for j in range(GATHERS):
            pltpu.async_copy(
                w_hbm.at[idx_v.at[pl.ds(c * ROWS_PER_CHUNK + j * 128, 128)]],
                rows_v.at[p, pl.ds(j * 128, 128)], sem_rows.at[p])
        pltpu.async_copy(
            w_hbm.at[tgt_v.at[pl.ds(c * C, C)]], trow_v.at[p],
            sem_tgt.at[p])

    def drain(c, p):
        for j in range(GATHERS):
            pltpu.make_async_copy(
                w_hbm.at[idx_v.at[pl.ds(c * ROWS_PER_CHUNK + j * 128, 128)]],
                rows_v.at[p, pl.ds(j * 128, 128)], sem_rows.at[p]).wait()
        pltpu.make_async_copy(
            w_hbm.at[tgt_v.at[pl.ds(c * C, C)]], trow_v.at[p],
            sem_tgt.at[p]).wait()

    issue(0, 0)

    def chunk_body(c, cos_sum):
        p = lax.rem(c, 2)

        @pl.when(c + 1 < NCH)
        def _():
            issue(c + 1, 1 - p)

        drain(c, p)

        for g in range(C // LANES):
            cnt = jnp.zeros((LANES,), jnp.int32)
            for l in range(L):
                ids = ctxt_v[l, pl.ds(c * C + g * LANES, LANES)]
                cnt = cnt + jnp.where(ids == jnp.int32(0),
                                      jnp.int32(1), jnp.int32(0))
            n0_v[pl.ds(g * LANES, LANES)] = cnt.astype(jnp.float32)

        def ebody(e, acc_cos):
            acc = [rows_v[p, e * L, pl.ds(16 * d, 16)] for d in range(4)]
            for l in range(1, L):
                for d in range(4):
                    acc[d] = acc[d] + rows_v[p, e * L + l, pl.ds(16 * d, 16)]
            n0 = jnp.full((LANES,), n0_v[pl.ds(e, LANES)][0], jnp.float32)
            inv = jnp.float32(1.0) / (jnp.float32(L) - n0 + jnp.float32(1e-6))
            pooled = [(acc[d] - n0 * w0_v[pl.ds(16 * d, 16)]) * inv
                      for d in range(4)]
            tg = [trow_v[p, e, pl.ds(16 * d, 16)] for d in range(4)]
            dv = pooled[0] * tg[0]
            av = pooled[0] * pooled[0]
            bv = tg[0] * tg[0]
            for d in range(1, 4):
                dv = dv + pooled[d] * tg[d]
                av = av + pooled[d] * pooled[d]
                bv = bv + tg[d] * tg[d]
            dots = jnp.full((LANES,), jnp.sum(dv), jnp.float32)
            na2 = jnp.full((LANES,), jnp.sum(av), jnp.float32)
            nb2 = jnp.full((LANES,), jnp.sum(bv), jnp.float32)
            na = _newton_sqrt(na2)
            nb = _newton_sqrt(nb2)
            cos = dots / (jnp.maximum(na, eps) * jnp.maximum(nb, eps))
            return acc_cos + cos

        return lax.fori_loop(0, C, ebody, cos_sum)

    total = lax.fori_loop(0, NCH, chunk_body,
                          jnp.zeros((LANES,), jnp.float32))

    cacc_v[...] = total
    pltpu.sync_copy(cacc_v, shared_v.at[sid])
    plsc.subcore_barrier()

    @pl.when(sid == 0)
    def _():
        pltpu.sync_copy(shared_v, red_v)
        tot = red_v[0, pl.ds(0, LANES)]
        for s in range(1, NS):
            tot = tot + red_v[s, pl.ds(0, LANES)]
        stage_v[...] = -tot / jnp.float32(B)
        pltpu.sync_copy(stage_v, out_hbm.at[cid])


@jax.jit
def _cbow_loss(ctxt, targets, Wt, wtail):
    mesh = plsc.VectorSubcoreMesh(
        core_axis_name="c", subcore_axis_name="s",
        num_cores=NC, num_subcores=NS)
    cp = pltpu.CompilerParams(
        needs_layout_passes=False, use_tc_tiling_on_sc=True)
    wpair = pl.kernel(
        _detile_body,
        out_type=jax.ShapeDtypeStruct((PAIR_ROWS, 128), jnp.float32),
        mesh=mesh,
        compiler_params=cp,
        scratch_types=[
            pltpu.VMEM((2, 64, 128), jnp.float32),
            pltpu.VMEM((2, 64, 128), jnp.float32),
            pltpu.SemaphoreType.DMA((2,)),
            pltpu.SemaphoreType.DMA((2,)),
        ],
    )(Wt, wtail)
    w2 = wpair.reshape(VOCAB, D)
    cp2 = pltpu.CompilerParams(
        needs_layout_passes=False, use_tc_tiling_on_sc=False)
    partial = pl.kernel(
        _cbow_body,
        out_type=jax.ShapeDtypeStruct((NC, LANES), jnp.float32),
        mesh=mesh,
        compiler_params=cp2,
        scratch_types=[
            pltpu.VMEM((IDX_LEN,), jnp.int32),
            pltpu.VMEM((L, BPW), jnp.int32),
            pltpu.VMEM((BPW,), jnp.int32),
            pltpu.VMEM((D,), jnp.float32),
            pltpu.VMEM((2, ROWS_PER_CHUNK, D), jnp.float32),
            pltpu.VMEM((2, C, D), jnp.float32),
            pltpu.VMEM((C + LANES,), jnp.float32),
            pltpu.VMEM((LANES,), jnp.float32),
            pltpu.VMEM((NS, LANES), jnp.float32),
            pltpu.VMEM((LANES,), jnp.float32),
            pltpu.VMEM_SHARED((NS, LANES), jnp.float32),
            pltpu.SemaphoreType.DMA((2,)),
            pltpu.SemaphoreType.DMA((2,)),
        ],
    )(w2, ctxt, targets)
    return partial[0, 0] + partial[1, 0]


def kernel(contexts, targets, W):
    wtail = W[VOCAB - 64:].reshape(32, 128)
    return _cbow_loss(contexts.T, targets, W.T, wtail)


# speedup vs baseline: 2.1767x; 1.9968x over previous
import jax
import jax.numpy as jnp
from jax import lax
from jax.experimental import pallas as pl
from jax.experimental.pallas import tpu as pltpu
from jax.experimental.pallas import tpu_sc as plsc

VOCAB = 1000000
D = 64
B = 16384
L = 20

NC = 2
NS = 16
LANES = 16
NW = NC * NS
BPW = B // NW
C = 32
NCH = BPW // C
IDX_LEN = BPW * L
ROWS_PER_CHUNK = C * L
GATHERS = ROWS_PER_CHUNK // 128


def _newton_sqrt(x):
    xs = jnp.maximum(x, jnp.float32(1e-30))
    i = plsc.bitcast(xs, jnp.int32)
    i = jnp.int32(0x5F3759DF) - (i >> 1)
    y = plsc.bitcast(i, jnp.float32)
    half = jnp.float32(0.5)
    threehalf = jnp.float32(1.5)
    for _ in range(3):
        y = y * (threehalf - half * xs * y * y)
    return xs * y


def _body(w_hbm, ctxt_hbm, tgt_hbm, out_hbm,
          idx_v, ctxt_v, tgt_v, w0_v, rows_v, trow_v,
          n0_v, cacc_v, red_v, stage_v, shared_v,
          sem_rows, sem_tgt):
    cid = lax.axis_index("c")
    sid = lax.axis_index("s")
    w = sid * NC + cid

    pltpu.sync_copy(ctxt_hbm.at[:, pl.ds(w * BPW, BPW)], ctxt_v)
    pltpu.sync_copy(tgt_hbm.at[pl.ds(w * BPW, BPW)], tgt_v)
    pltpu.sync_copy(w_hbm.at[0], w0_v)

    lane = jnp.arange(LANES, dtype=jnp.int32)

    def tbody(g, carry):
        base = (g * LANES + lane) * L
        for l in range(L):
            ids = ctxt_v[l, pl.ds(g * LANES, LANES)]
            plsc.store_scatter(idx_v, [base + l], ids)
        return carry

    lax.fori_loop(0, BPW // LANES, tbody, 0)

    eps = jnp.float32(1e-8)

    def issue(c, p):
        for j in range(GATHERS):
            pltpu.async_copy(
                w_hbm.at[idx_v.at[pl.ds(c * ROWS_PER_CHUNK + j * 128, 128)]],
                rows_v.at[p, pl.ds(j * 128, 128)], sem_rows.at[p])
        pltpu.async_copy(
            w_hbm.at[tgt_v.at[pl.ds(c * C, C)]], trow_v.at[p],
            sem_tgt.at[p])

    def drain(c, p):
        for j in range(GATHERS):
            pltpu.make_async_copy(
                w_hbm.at[idx_v.at[pl.ds(c * ROWS_PER_CHUNK + j * 128, 128)]],
                rows_v.at[p, pl.ds(j * 128, 128)], sem_rows.at[p]).wait()
        pltpu.make_async_copy(
            w_hbm.at[tgt_v.at[pl.ds(c * C, C)]], trow_v.at[p],
            sem_tgt.at[p]).wait()

    issue(0, 0)

    def chunk_body(c, cos_sum):
        p = lax.rem(c, 2)

        @pl.when(c + 1 < NCH)
        def _():
            issue(c + 1, 1 - p)

        drain(c, p)

        for g in range(C // LANES):
            cnt = jnp.zeros((LANES,), jnp.int32)
            for l in range(L):
                ids = ctxt_v[l, pl.ds(c * C + g * LANES, LANES)]
                cnt = cnt + jnp.where(ids == jnp.int32(0),
                                      jnp.int32(1), jnp.int32(0))
            n0_v[pl.ds(g * LANES, LANES)] = cnt.astype(jnp.float32)

        def ebody(e, acc_cos):
            acc = [rows_v[p, e * L, pl.ds(16 * d, 16)] for d in range(4)]
            for l in range(1, L):
                for d in range(4):
                    acc[d] = acc[d] + rows_v[p, e * L + l, pl.ds(16 * d, 16)]
            n0 = jnp.full((LANES,), n0_v[pl.ds(e, LANES)][0], jnp.float32)
            inv = jnp.float32(1.0) / (jnp.float32(L) - n0 + jnp.float32(1e-6))
            pooled = [(acc[d] - n0 * w0_v[pl.ds(16 * d, 16)]) * inv
                      for d in range(4)]
            tg = [trow_v[p, e, pl.ds(16 * d, 16)] for d in range(4)]
            dv = pooled[0] * tg[0]
            av = pooled[0] * pooled[0]
            bv = tg[0] * tg[0]
            for d in range(1, 4):
                dv = dv + pooled[d] * tg[d]
                av = av + pooled[d] * pooled[d]
                bv = bv + tg[d] * tg[d]
            dots = jnp.full((LANES,), jnp.sum(dv), jnp.float32)
            na2 = jnp.full((LANES,), jnp.sum(av), jnp.float32)
            nb2 = jnp.full((LANES,), jnp.sum(bv), jnp.float32)
            na = _newton_sqrt(na2)
            nb = _newton_sqrt(nb2)
            cos = dots / (jnp.maximum(na, eps) * jnp.maximum(nb, eps))
            return acc_cos + cos

        return lax.fori_loop(0, C, ebody, cos_sum)

    total = lax.fori_loop(0, NCH, chunk_body,
                          jnp.zeros((LANES,), jnp.float32))

    cacc_v[...] = total
    pltpu.sync_copy(cacc_v, shared_v.at[sid])
    plsc.subcore_barrier()

    @pl.when(sid == 0)
    def _():
        pltpu.sync_copy(shared_v, red_v)
        tot = red_v[0, pl.ds(0, LANES)]
        for s in range(1, NS):
            tot = tot + red_v[s, pl.ds(0, LANES)]
        stage_v[...] = -tot / jnp.float32(B)
        pltpu.sync_copy(stage_v, out_hbm.at[cid])


@jax.jit
def _cbow_loss(ctxt, targets, W):
    mesh = plsc.VectorSubcoreMesh(
        core_axis_name="c", subcore_axis_name="s",
        num_cores=NC, num_subcores=NS)
    partial = pl.kernel(
        _body,
        out_type=jax.ShapeDtypeStruct((NC, LANES), jnp.float32),
        mesh=mesh,
        compiler_params=pltpu.CompilerParams(
            needs_layout_passes=False, use_tc_tiling_on_sc=False),
        scratch_types=[
            pltpu.VMEM((IDX_LEN,), jnp.int32),
            pltpu.VMEM((L, BPW), jnp.int32),
            pltpu.VMEM((BPW,), jnp.int32),
            pltpu.VMEM((D,), jnp.float32),
            pltpu.VMEM((2, ROWS_PER_CHUNK, D), jnp.float32),
            pltpu.VMEM((2, C, D), jnp.float32),
            pltpu.VMEM((C + LANES,), jnp.float32),
            pltpu.VMEM((LANES,), jnp.float32),
            pltpu.VMEM((NS, LANES), jnp.float32),
            pltpu.VMEM((LANES,), jnp.float32),
            pltpu.VMEM_SHARED((NS, LANES), jnp.float32),
            pltpu.SemaphoreType.DMA((2,)),
            pltpu.SemaphoreType.DMA((2,)),
        ],
    )(W, ctxt, targets)
    return partial[0, 0] + partial[1, 0]


def kernel(contexts, targets, W):
    return _cbow_loss(contexts.T, targets, W)
